# Initial kernel scaffold; baseline (speedup 1.0000x reference)
import jax
import jax.numpy as jnp
from jax.experimental import pallas as pl


def kernel(x, mu, log_sigma, eps):
    raise NotImplementedError("write your pallas kernel here")


# speedup vs baseline: 1.0070x; 1.0070x over previous
import functools

import jax
import jax.numpy as jnp
from jax import lax
from jax.experimental import pallas as pl
from jax.experimental.pallas import tpu as pltpu

INPUT_DIM = 4096
RANK = 2049
ONE_BITS = 0x3F800000


def _mask_kernel(mu_ref, ls_ref, eps_ref, mask_ref, fmask_ref):
    z = mu_ref[...] + eps_ref[...] * jnp.exp(0.5 * ls_ref[...])
    m = 1.0 / (1.0 + jnp.exp(-z))
    mask_ref[...] = m
    keys = lax.bitcast_convert_type(m, jnp.int32)

    def body(_, carry):
        lo, hi = carry
        mid = lo + (hi - lo) // 2
        cnt = jnp.sum((keys <= mid).astype(jnp.int32))
        pred = cnt >= RANK
        return (jnp.where(pred, lo, mid + 1), jnp.where(pred, mid, hi))

    lo, hi = lax.fori_loop(0, 31, body, (jnp.int32(0), jnp.int32(ONE_BITS)))
    fmask_ref[...] = (keys > lo).astype(jnp.float32)


def _mul_kernel(x_ref, fm_ref, o_ref):
    o_ref[...] = x_ref[...] * fm_ref[...]


@functools.partial(jax.jit, static_argnames=("rows",))
def _run(x, mu, log_sigma, eps, rows=256):
    mask2d, fmask2d = pl.pallas_call(
        _mask_kernel,
        out_shape=[
            jax.ShapeDtypeStruct((32, 128), jnp.float32),
            jax.ShapeDtypeStruct((32, 128), jnp.float32),
        ],
    )(mu.reshape(32, 128), log_sigma.reshape(32, 128), eps.reshape(32, 128))

    xf = x.reshape(-1, INPUT_DIM)
    n = xf.shape[0]
    out = pl.pallas_call(
        _mul_kernel,
        grid=(n // rows,),
        in_specs=[
            pl.BlockSpec((rows, INPUT_DIM), lambda i: (i, 0)),
            pl.BlockSpec((1, INPUT_DIM), lambda i: (0, 0)),
        ],
        out_specs=pl.BlockSpec((rows, INPUT_DIM), lambda i: (i, 0)),
        out_shape=jax.ShapeDtypeStruct((n, INPUT_DIM), jnp.float32),
        compiler_params=pltpu.CompilerParams(
            dimension_semantics=("arbitrary",),
        ),
    )(xf, fmask2d.reshape(1, INPUT_DIM))
    return out.reshape(x.shape), mask2d.reshape(INPUT_DIM)


def kernel(x, mu, log_sigma, eps):
    return _run(x, mu, log_sigma, eps)
